# baseline (speedup 1.0000x reference)
import functools

import jax
import jax.numpy as jnp
from jax import lax
from jax.experimental import pallas as pl
from jax.experimental.pallas import tpu as pltpu
from jax.experimental.pallas import tpu_sc as plsc

VOCAB = 1000000
DIM = 32
PAD = 0

NC = 2
NS = 16
L = 16
NW = NC * NS

NBATCH = 4096
SEQ = 200
ST = SEQ // 8

NBUF = 4
NOB = 4
AHEAD = 3


def _body(t4_hbm, xq_hbm, out_hbm, idx_v, rlist, rows2, outt2, gsem, osem):
    wid = lax.axis_index("s") * NC + lax.axis_index("c")

    pltpu.sync_copy(xq_hbm.at[:, wid], idx_v)

    iota16 = lax.iota(jnp.int32, L)
    rowvecs = [blv * L + iota16 for blv in range(8)]

    def fire_gather(s, b):
        st, si = s // 8, s % 8
        for blv in range(8):
            iv = idx_v[st, si, pl.ds(blv * L, L)]
            rlist[b, pl.ds(blv * L, L)] = lax.shift_right_logical(iv, 2)
        pltpu.async_copy(
            t4_hbm.at[rlist.at[b]], rows2.at[b], gsem.at[b]
        )

    def drain_gather(b):
        pltpu.make_async_copy(
            t4_hbm.at[pl.ds(0, 128)],
            rows2.at[b],
            gsem.at[b],
        ).wait()

    def fire_out(s, b):
        pltpu.async_copy(
            outt2.at[b, :, :, pl.ds(0, 128)],
            out_hbm.at[s].at[:, wid],
            osem.at[b],
        )

    def wait_out(b):
        pltpu.make_async_copy(
            outt2.at[b, :, :, pl.ds(0, 128)],
            out_hbm.at[0].at[:, wid],
            osem.at[b],
        ).wait()

    def compute(s, gb, ob):
        st, si = s // 8, s % 8
        colb = []
        fv = []
        for blv in range(8):
            iv = idx_v[st, si, pl.ds(blv * L, L)]
            colb.append(lax.shift_left(jnp.bitwise_and(iv, 3), 5))
            fv.append(jnp.where(iv == PAD, 0.0, 1.0))

        fgc = [
            lax.shift_right_logical(h * L + iota16, 3) for h in range(2)
        ]
        fic = jnp.bitwise_and(iota16, 7)
        for blv in range(8):
            for bl_in in range(L):
                bl = blv * L + bl_in
                lane = jnp.full((L,), bl_in, jnp.int32)
                cb = colb[blv].at[lane].get(mode="promise_in_bounds")
                fb = fv[blv].at[lane].get(mode="promise_in_bounds")
                civ = cb + iota16
                blc = jnp.full((L,), bl, jnp.int32)
                rowsplat = blc
                for h in range(2):
                    v = plsc.load_gather(
                        rows2.at[gb], [rowsplat, civ + h * L]
                    )
                    plsc.store_scatter(
                        outt2.at[ob], [fgc[h], fic, blc], v * fb
                    )

    for s in range(AHEAD):
        fire_gather(s, s % NBUF)

    def block(t, carry):
        s0 = NBUF * t
        for j in range(NBUF):
            s = s0 + j
            gb = j
            ob = j % NOB

            @pl.when(s >= NOB)
            def _():
                wait_out(ob)

            @pl.when(s + AHEAD < SEQ)
            def _():
                fire_gather(s + AHEAD, (j + AHEAD) % NBUF)

            drain_gather(gb)
            compute(s, gb, ob)
            fire_out(s, ob)
        return carry

    lax.fori_loop(0, SEQ // NBUF, block, 0)
    for b in range(NOB):
        wait_out(b)


@functools.partial(jax.jit, static_argnames=())
def _embed(t4, xq):
    f = pl.kernel(
        _body,
        out_type=jax.ShapeDtypeStruct((SEQ, 4, NW, 8, 128), jnp.float32),
        mesh=plsc.VectorSubcoreMesh(
            core_axis_name="c", subcore_axis_name="s",
            num_cores=NC, num_subcores=NS,
        ),
        scratch_types=[
            pltpu.VMEM((ST, 8, 128), jnp.int32),
            pltpu.VMEM((NBUF, 128), jnp.int32),
            pltpu.VMEM((NBUF, 128, 128), jnp.float32),
            pltpu.VMEM((NOB, 4, 8, 133), jnp.float32),
            pltpu.SemaphoreType.DMA((NBUF,)),
            pltpu.SemaphoreType.DMA((NOB,)),
        ],
        compiler_params=pltpu.CompilerParams(
            use_tc_tiling_on_sc=False, needs_layout_passes=False,
        ),
    )
    return f(t4, xq)


def kernel(X, emb_table):
    xq = (
        X.astype(jnp.int32)
        .reshape(NW, 128, ST, 8)
        .transpose(2, 0, 3, 1)
    )
    t4 = emb_table.reshape(VOCAB // 4, 128)
    out5 = _embed(t4, xq)
    return out5.transpose(2, 4, 0, 1, 3).reshape(NBATCH, SEQ, DIM)


# speedup vs baseline: 1.1330x; 1.1330x over previous
import functools

import jax
import jax.numpy as jnp
from jax import lax
from jax.experimental import pallas as pl
from jax.experimental.pallas import tpu as pltpu
from jax.experimental.pallas import tpu_sc as plsc

VOCAB = 1000000
DIM = 32
PAD = 0

NC = 2
NS = 16
L = 16
NW = NC * NS

NBATCH = 4096
SEQ = 200
ST = SEQ // 8

NBUF = 4
NOB = 4
AHEAD = 3


def _body(t4_hbm, xq_hbm, out_hbm, idx_v, rlist, rows2, outt2, gsem, osem):
    wid = lax.axis_index("s") * NC + lax.axis_index("c")

    pltpu.sync_copy(xq_hbm.at[:, wid], idx_v)

    iota16 = lax.iota(jnp.int32, L)
    rowvecs = [blv * L + iota16 for blv in range(8)]

    def fire_gather(s, b):
        st, si = s // 8, s % 8
        for blv in range(8):
            iv = idx_v[st, si, pl.ds(blv * L, L)]
            rlist[b, pl.ds(blv * L, L)] = lax.shift_right_logical(iv, 2)
        pltpu.async_copy(
            t4_hbm.at[rlist.at[b]], rows2.at[b], gsem.at[b]
        )

    def drain_gather(b):
        pltpu.make_async_copy(
            t4_hbm.at[pl.ds(0, 128)],
            rows2.at[b],
            gsem.at[b],
        ).wait()

    def fire_out(s, b):
        pltpu.async_copy(
            outt2.at[b], out_hbm.at[s].at[:, wid], osem.at[b]
        )

    def wait_out(b):
        pltpu.make_async_copy(
            outt2.at[b], out_hbm.at[0].at[:, wid], osem.at[b]
        ).wait()

    def compute(s, gb, ob):
        st, si = s // 8, s % 8
        colb = []
        fv = []
        for blv in range(8):
            iv = idx_v[st, si, pl.ds(blv * L, L)]
            colb.append(lax.shift_left(jnp.bitwise_and(iv, 3), 5))
            fv.append(jnp.where(iv == PAD, 0.0, 1.0))

        def load_batch(fg, bh):
            batch = []
            for fi in range(8):
                for blv in range(bh * 2, bh * 2 + 2):
                    v = plsc.load_gather(
                        rows2.at[gb],
                        [rowvecs[blv], colb[blv] + (fg * 8 + fi)],
                    )
                    batch.append((fg, fi, blv, v))
            return batch

        def store_batch(batch):
            for fg, fi, blv, v in batch:
                outt2[ob, fg, fi, pl.ds(blv * L, L)] = v * fv[blv]

        steps = [(fg, bh) for fg in range(4) for bh in range(4)]
        prev = load_batch(*steps[0])
        for step in steps[1:]:
            cur = load_batch(*step)
            store_batch(prev)
            prev = cur
        store_batch(prev)

    for s in range(AHEAD):
        fire_gather(s, s % NBUF)

    def block(t, carry):
        s0 = NBUF * t
        for j in range(NBUF):
            s = s0 + j
            gb = j
            ob = j % NOB

            @pl.when(s >= NOB)
            def _():
                wait_out(ob)

            @pl.when(s + AHEAD < SEQ)
            def _():
                fire_gather(s + AHEAD, (j + AHEAD) % NBUF)

            drain_gather(gb)
            compute(s, gb, ob)
            fire_out(s, ob)
        return carry

    lax.fori_loop(0, SEQ // NBUF, block, 0)
    for b in range(NOB):
        wait_out(b)


@functools.partial(jax.jit, static_argnames=())
def _embed(t4, xq):
    f = pl.kernel(
        _body,
        out_type=jax.ShapeDtypeStruct((SEQ, 4, NW, 8, 128), jnp.float32),
        mesh=plsc.VectorSubcoreMesh(
            core_axis_name="c", subcore_axis_name="s",
            num_cores=NC, num_subcores=NS,
        ),
        scratch_types=[
            pltpu.VMEM((ST, 8, 128), jnp.int32),
            pltpu.VMEM((NBUF, 128), jnp.int32),
            pltpu.VMEM((NBUF, 128, 128), jnp.float32),
            pltpu.VMEM((NOB, 4, 8, 128), jnp.float32),
            pltpu.SemaphoreType.DMA((NBUF,)),
            pltpu.SemaphoreType.DMA((NOB,)),
        ],
        compiler_params=pltpu.CompilerParams(
            use_tc_tiling_on_sc=False, needs_layout_passes=False,
        ),
    )
    return f(t4, xq)


def kernel(X, emb_table):
    xq = (
        X.astype(jnp.int32)
        .reshape(NW, 128, ST, 8)
        .transpose(2, 0, 3, 1)
    )
    t4 = emb_table.reshape(VOCAB // 4, 128)
    out5 = _embed(t4, xq)
    return out5.transpose(2, 4, 0, 1, 3).reshape(NBATCH, SEQ, DIM)
